# baseline (speedup 1.0000x reference)
import functools

import jax
import jax.numpy as jnp
from jax import lax
from jax.experimental import pallas as pl
from jax.experimental.pallas import tpu as pltpu
from jax.experimental.pallas import tpu_sc as plsc

BATCH = 16384
NUM_CORES = 2
NUM_SUBCORES = 16
LANES = 16
NUM_WORKERS = NUM_CORES * NUM_SUBCORES
B_PER_W = BATCH // NUM_WORKERS


def _sigmoid(x):
    return 1.0 / (1.0 + jnp.exp(-x))


def _build_sc_kernel():
    mesh = plsc.VectorSubcoreMesh(core_axis_name="c", subcore_axis_name="s")

    @functools.partial(
        pl.kernel,
        mesh=mesh,
        out_type=jax.ShapeDtypeStruct((BATCH,), jnp.float32),
        scratch_types=[
            pltpu.VMEM((B_PER_W,), jnp.int32),
            pltpu.VMEM((B_PER_W,), jnp.int32),
            pltpu.VMEM((B_PER_W,), jnp.float32),
            pltpu.VMEM((B_PER_W,), jnp.float32),
            pltpu.VMEM((B_PER_W,), jnp.float32),
            pltpu.VMEM((B_PER_W,), jnp.float32),
            pltpu.SemaphoreType.DMA,
            pltpu.SemaphoreType.DMA,
        ],
    )
    def sc_kernel(stu_id_hbm, exer_id_hbm, stu_emb_hbm, kdiff_hbm, edisc_hbm,
                  out_hbm, sidx_v, eidx_v, s_v, k_v, d_v, o_v, sem, isem):
        wid = lax.axis_index("s") * NUM_CORES + lax.axis_index("c")
        base = wid * B_PER_W
        ci_e = pltpu.async_copy(exer_id_hbm.at[pl.ds(base, B_PER_W)], eidx_v, isem)
        ci_s = pltpu.async_copy(stu_id_hbm.at[pl.ds(base, B_PER_W)], sidx_v, isem)
        ci_e.wait()
        c_k = pltpu.async_copy(kdiff_hbm.at[eidx_v], k_v, sem)
        c_d = pltpu.async_copy(edisc_hbm.at[eidx_v], d_v, sem)
        ci_s.wait()
        c_s = pltpu.async_copy(stu_emb_hbm.at[sidx_v], s_v, sem)

        c_k.wait()
        c_d.wait()
        for i in range(B_PER_W // LANES):
            sl = pl.ds(i * LANES, LANES)
            k_v[sl] = _sigmoid(k_v[sl])
            d_v[sl] = _sigmoid(d_v[sl]) * 10.0
        c_s.wait()
        for i in range(B_PER_W // LANES):
            sl = pl.ds(i * LANES, LANES)
            o_v[sl] = _sigmoid(d_v[sl] * (_sigmoid(s_v[sl]) - k_v[sl]))
        pltpu.sync_copy(o_v, out_hbm.at[pl.ds(base, B_PER_W)])

    return sc_kernel


_SC_KERNEL = _build_sc_kernel()


@jax.jit
def kernel(stu_id, exer_id, student_emb, k_difficulty, e_discrimination):
    out = _SC_KERNEL(
        stu_id.astype(jnp.int32),
        exer_id.astype(jnp.int32),
        student_emb.reshape(-1),
        k_difficulty.reshape(-1),
        e_discrimination.reshape(-1),
    )
    return out.reshape(BATCH, 1)


# speedup vs baseline: 1.0316x; 1.0316x over previous
import functools

import jax
import jax.numpy as jnp
from jax import lax
from jax.experimental import pallas as pl
from jax.experimental.pallas import tpu as pltpu
from jax.experimental.pallas import tpu_sc as plsc

BATCH = 16384
NUM_CORES = 2
NUM_SUBCORES = 16
LANES = 16
NUM_WORKERS = NUM_CORES * NUM_SUBCORES
B_PER_W = BATCH // NUM_WORKERS


def _sigmoid(x):
    return 1.0 / (1.0 + jnp.exp(-x))


def _build_sc_kernel():
    mesh = plsc.VectorSubcoreMesh(core_axis_name="c", subcore_axis_name="s")

    @functools.partial(
        pl.kernel,
        mesh=mesh,
        out_type=jax.ShapeDtypeStruct((BATCH,), jnp.float32),
        scratch_types=[
            pltpu.VMEM((B_PER_W,), jnp.int32),
            pltpu.VMEM((B_PER_W,), jnp.int32),
            pltpu.VMEM((B_PER_W,), jnp.float32),
            pltpu.VMEM((B_PER_W,), jnp.float32),
            pltpu.VMEM((B_PER_W,), jnp.float32),
            pltpu.VMEM((B_PER_W,), jnp.float32),
            pltpu.SemaphoreType.DMA,
            pltpu.SemaphoreType.DMA,
        ],
    )
    def sc_kernel(stu_id_hbm, exer_id_hbm, stu_emb_hbm, kdiff_hbm, edisc_hbm,
                  out_hbm, sidx_v, eidx_v, s_v, k_v, d_v, o_v, sem, isem):
        wid = lax.axis_index("s") * NUM_CORES + lax.axis_index("c")
        base = wid * B_PER_W
        ci_e = pltpu.async_copy(exer_id_hbm.at[pl.ds(base, B_PER_W)], eidx_v, isem)
        ci_s = pltpu.async_copy(stu_id_hbm.at[pl.ds(base, B_PER_W)], sidx_v, isem)
        ci_e.wait()
        c_k = pltpu.async_copy(kdiff_hbm.at[eidx_v], k_v, sem)
        c_d = pltpu.async_copy(edisc_hbm.at[eidx_v], d_v, sem)
        ci_s.wait()
        c_s = pltpu.async_copy(stu_emb_hbm.at[sidx_v], s_v, sem)

        c_k.wait()
        c_d.wait()
        c_s.wait()

        def body(i, carry):
            sl = pl.ds(i * LANES, LANES)
            es = jnp.exp(-s_v[sl])
            ek = jnp.exp(-k_v[sl])
            ed = jnp.exp(-d_v[sl])
            t = (10.0 * (ek - es)) / ((1.0 + es) * ((1.0 + ek) * (1.0 + ed)))
            o_v[sl] = 1.0 / (1.0 + jnp.exp(-t))
            return carry

        lax.fori_loop(0, B_PER_W // LANES, body, 0)
        pltpu.sync_copy(o_v, out_hbm.at[pl.ds(base, B_PER_W)])

    return sc_kernel


_SC_KERNEL = _build_sc_kernel()


@jax.jit
def kernel(stu_id, exer_id, student_emb, k_difficulty, e_discrimination):
    out = _SC_KERNEL(
        stu_id.astype(jnp.int32),
        exer_id.astype(jnp.int32),
        student_emb.reshape(-1),
        k_difficulty.reshape(-1),
        e_discrimination.reshape(-1),
    )
    return out.reshape(BATCH, 1)
